# baseline (speedup 1.0000x reference)
import functools

import jax
import jax.numpy as jnp
from jax import lax
from jax.experimental import pallas as pl
from jax.experimental.pallas import tpu as pltpu
from jax.experimental.pallas import tpu_sc as plsc

NC = 2
NS = 16
NW = NC * NS
NBUF = 6
CHUNK = 120


@functools.cache
def _build(n: int, d: int, dtype):
    per_w = n // NW
    base_chunks = (per_w // CHUNK) * CHUNK
    max_rows = base_chunks + 8
    n_chunks = base_chunks // CHUNK
    assert base_chunks == per_w // 8 * 8
    nbuf = min(NBUF, n_chunks)
    mesh = plsc.VectorSubcoreMesh(core_axis_name="c", subcore_axis_name="s")

    @functools.partial(
        pl.kernel,
        mesh=mesh,
        out_type=jax.ShapeDtypeStruct((n, d), dtype),
        scratch_types=[
            pltpu.VMEM((max_rows,), jnp.int32),
            pltpu.VMEM((nbuf, CHUNK, d), dtype),
            [pltpu.SemaphoreType.DMA] * nbuf,
            [pltpu.SemaphoreType.DMA] * nbuf,
            pltpu.SemaphoreType.DMA,
        ],
        compiler_params=pltpu.CompilerParams(use_tc_tiling_on_sc=False),
    )
    def gather_kernel(idx_hbm, table_hbm, out_hbm, idx_v, buf_v, gsems, ssems,
                      tsem):
        wid = lax.axis_index("s") * NC + lax.axis_index("c")
        base = pl.multiple_of(wid * per_w // 8 * 8, 8)
        nxt = pl.multiple_of((wid + 1) * per_w // 8 * 8, 8)
        rows = nxt - base
        head = min(nbuf - 1, n_chunks) * CHUNK
        pltpu.sync_copy(idx_hbm.at[pl.ds(base, head)],
                        idx_v.at[pl.ds(0, head)])

        def gather_start(c):
            return pltpu.async_copy(
                table_hbm.at[idx_v.at[pl.ds(c * CHUNK, CHUNK)]],
                buf_v.at[c % nbuf],
                gsems[c % nbuf],
            )

        def store_start(c):
            return pltpu.async_copy(
                buf_v.at[c % nbuf], out_hbm.at[pl.ds(base + c * CHUNK, CHUNK)],
                ssems[c % nbuf],
            )

        gh = [None] * n_chunks
        sh = [None] * n_chunks
        for c in range(nbuf - 1):
            gh[c] = gather_start(c)
        if head < max_rows:
            pltpu.sync_copy(idx_hbm.at[pl.ds(base + head, max_rows - head)],
                            idx_v.at[pl.ds(head, max_rows - head)])
        for c in range(n_chunks):
            if c + nbuf - 1 < n_chunks:
                if c >= 1:
                    sh[c - 1].wait()
                gh[c + nbuf - 1] = gather_start(c + nbuf - 1)
            gh[c].wait()
            sh[c] = store_start(c)
        for c in range(max(0, n_chunks - nbuf), n_chunks):
            sh[c].wait()

        @pl.when(rows > base_chunks)
        def _tail():
            pltpu.async_copy(
                table_hbm.at[idx_v.at[pl.ds(base_chunks, 8)]],
                buf_v.at[0, pl.ds(0, 8)],
                tsem,
            ).wait()
            pltpu.async_copy(
                buf_v.at[0, pl.ds(0, 8)],
                out_hbm.at[pl.ds(base + base_chunks, 8)],
                tsem,
            ).wait()

    return gather_kernel


def kernel(x, table):
    n = x.shape[0]
    d = table.shape[1]
    assert n % NW == 0 and n % 8 == 0, n
    return _build(n, d, table.dtype)(x.astype(jnp.int32), table)


# speedup vs baseline: 1.0060x; 1.0060x over previous
import functools

import jax
import jax.numpy as jnp
from jax import lax
from jax.experimental import pallas as pl
from jax.experimental.pallas import tpu as pltpu
from jax.experimental.pallas import tpu_sc as plsc

NC = 2
NS = 16
NW = NC * NS
NBUF = 6
CHUNK = 120


@functools.cache
def _build(n: int, d: int, dtype):
    per_w = n // NW
    base_chunks = (per_w // CHUNK) * CHUNK
    max_rows = base_chunks + 8
    n_chunks = base_chunks // CHUNK
    assert base_chunks == per_w // 8 * 8
    nbuf = min(NBUF, n_chunks)
    mesh = plsc.VectorSubcoreMesh(core_axis_name="c", subcore_axis_name="s")

    @functools.partial(
        pl.kernel,
        mesh=mesh,
        out_type=jax.ShapeDtypeStruct((n, d), dtype),
        scratch_types=[
            pltpu.VMEM((max_rows,), jnp.int32),
            pltpu.VMEM((nbuf, CHUNK, d), dtype),
            pltpu.VMEM((8, d), dtype),
            [pltpu.SemaphoreType.DMA] * nbuf,
            [pltpu.SemaphoreType.DMA] * nbuf,
            pltpu.SemaphoreType.DMA,
        ],
        compiler_params=pltpu.CompilerParams(use_tc_tiling_on_sc=False),
    )
    def gather_kernel(idx_hbm, table_hbm, out_hbm, idx_v, buf_v, tail_v,
                      gsems, ssems, tsem):
        wid = lax.axis_index("s") * NC + lax.axis_index("c")
        base = pl.multiple_of(wid * per_w // 8 * 8, 8)
        nxt = pl.multiple_of((wid + 1) * per_w // 8 * 8, 8)
        rows = nxt - base
        head = min(nbuf - 1, n_chunks) * CHUNK
        pltpu.sync_copy(idx_hbm.at[pl.ds(base, head)],
                        idx_v.at[pl.ds(0, head)])

        def gather_start(c):
            return pltpu.async_copy(
                table_hbm.at[idx_v.at[pl.ds(c * CHUNK, CHUNK)]],
                buf_v.at[c % nbuf],
                gsems[c % nbuf],
            )

        def store_start(c):
            return pltpu.async_copy(
                buf_v.at[c % nbuf], out_hbm.at[pl.ds(base + c * CHUNK, CHUNK)],
                ssems[c % nbuf],
            )

        gh = [None] * n_chunks
        sh = [None] * n_chunks
        for c in range(nbuf - 1):
            gh[c] = gather_start(c)
        if head < max_rows:
            pltpu.sync_copy(idx_hbm.at[pl.ds(base + head, max_rows - head)],
                            idx_v.at[pl.ds(head, max_rows - head)])

        has_tail = rows > base_chunks

        @pl.when(has_tail)
        def _tail_gather():
            pltpu.make_async_copy(
                table_hbm.at[idx_v.at[pl.ds(base_chunks, 8)]],
                tail_v,
                tsem,
            ).start()

        for c in range(n_chunks):
            if c + nbuf - 1 < n_chunks:
                if c >= 1:
                    sh[c - 1].wait()
                gh[c + nbuf - 1] = gather_start(c + nbuf - 1)
            gh[c].wait()
            sh[c] = store_start(c)
        for c in range(max(0, n_chunks - nbuf), n_chunks):
            sh[c].wait()

        @pl.when(has_tail)
        def _tail_store():
            pltpu.make_async_copy(
                table_hbm.at[idx_v.at[pl.ds(base_chunks, 8)]],
                tail_v,
                tsem,
            ).wait()
            pltpu.async_copy(
                tail_v, out_hbm.at[pl.ds(base + base_chunks, 8)], tsem
            ).wait()

    return gather_kernel


def kernel(x, table):
    n = x.shape[0]
    d = table.shape[1]
    assert n % NW == 0 and n % 8 == 0, n
    return _build(n, d, table.dtype)(x.astype(jnp.int32), table)


# speedup vs baseline: 1.0093x; 1.0033x over previous
import functools

import jax
import jax.numpy as jnp
from jax import lax
from jax.experimental import pallas as pl
from jax.experimental.pallas import tpu as pltpu
from jax.experimental.pallas import tpu_sc as plsc

NC = 2
NS = 16
NW = NC * NS
NBUF = 6


def _chunk_rows(rows_per_worker: int) -> int:
    best = 1
    for c in range(1, 257):
        if rows_per_worker % c == 0:
            best = c
    return best


@functools.cache
def _build(rows_per_worker: int, chunk: int, d: int, dtype):
    n_chunks = rows_per_worker // chunk
    nbuf = min(NBUF, n_chunks)
    mesh = plsc.VectorSubcoreMesh(core_axis_name="c", subcore_axis_name="s")

    @functools.partial(
        pl.kernel,
        mesh=mesh,
        out_type=jax.ShapeDtypeStruct((NW * rows_per_worker, d), dtype),
        scratch_types=[
            pltpu.VMEM((n_chunks, chunk), jnp.int32),
            pltpu.VMEM((nbuf, chunk, d), dtype),
            [pltpu.SemaphoreType.DMA] * nbuf,
            [pltpu.SemaphoreType.DMA] * nbuf,
        ],
        compiler_params=pltpu.CompilerParams(use_tc_tiling_on_sc=False),
    )
    def gather_kernel(idx_hbm, table_hbm, out_hbm, idx_v, buf_v, gsems, ssems):
        wid = lax.axis_index("s") * NC + lax.axis_index("c")
        base = wid * rows_per_worker
        head = min(nbuf - 1, n_chunks)
        pltpu.sync_copy(
            idx_hbm.at[wid, pl.ds(0, head)], idx_v.at[pl.ds(0, head)]
        )

        def gather_start(c):
            return pltpu.async_copy(
                table_hbm.at[idx_v.at[c]], buf_v.at[c % nbuf], gsems[c % nbuf]
            )

        def store_start(c):
            return pltpu.async_copy(
                buf_v.at[c % nbuf], out_hbm.at[pl.ds(base + c * chunk, chunk)],
                ssems[c % nbuf],
            )

        gh = [None] * n_chunks
        sh = [None] * n_chunks
        for c in range(head):
            gh[c] = gather_start(c)
        if head < n_chunks:
            pltpu.sync_copy(
                idx_hbm.at[wid, pl.ds(head, n_chunks - head)],
                idx_v.at[pl.ds(head, n_chunks - head)],
            )
        for c in range(n_chunks):
            if c + nbuf - 1 < n_chunks:
                if c >= 1:
                    sh[c - 1].wait()
                gh[c + nbuf - 1] = gather_start(c + nbuf - 1)
            gh[c].wait()
            sh[c] = store_start(c)
        for c in range(max(0, n_chunks - nbuf), n_chunks):
            sh[c].wait()

    return gather_kernel


def kernel(x, table):
    n = x.shape[0]
    d = table.shape[1]
    assert n % NW == 0, n
    rows_per_worker = n // NW
    chunk = _chunk_rows(rows_per_worker)
    xi = x.astype(jnp.int32).reshape(NW, rows_per_worker // chunk, chunk)
    return _build(rows_per_worker, chunk, d, table.dtype)(xi, table)
